# baseline (speedup 1.0000x reference)
import functools

import jax
import jax.numpy as jnp
from jax import lax
from jax.experimental import pallas as pl
from jax.experimental.pallas import tpu as pltpu
from jax.experimental.pallas import tpu_sc as plsc

D = 64
DP = 96
GW = 32
HIST = 50
HISTP = 56
HISTQ = 104
LANES = 16
NC = 2
NS = 16
NW = NC * NS
NBUF = 4


def _tower_body(xp_hbm, ms_hbm, table_hbm, out_hbm,
                xp_v, ms_v, rows0, rows1, rows2, rows3, out_v,
                sem0, sem1, sem2, sem3, bpw):
    wid = lax.axis_index("s") * NC + lax.axis_index("c")
    rows = (rows0, rows1, rows2, rows3)
    sems = (sem0, sem1, sem2, sem3)
    out_base = wid * bpw * D

    pltpu.sync_copy(xp_hbm.at[pl.ds(wid * bpw * HISTQ, bpw * HISTQ)], xp_v)
    pltpu.sync_copy(ms_hbm.at[pl.ds(wid * bpw * HISTP, bpw * HISTP)], ms_v)

    group_starts = (0, 16, 32, 40)

    def issue(b, ph):
        pltpu.async_copy(table_hbm.at[xp_v.at[pl.ds(b * HISTQ, 2 * HIST)]],
                         rows[ph], sems[ph])

    for ph in range(NBUF):
        issue(ph, ph)

    def compute_row(b, rbuf):
        acc = [jnp.zeros((LANES,), jnp.float32) for _ in range(D // LANES)]
        cnt = jnp.zeros((LANES,), jnp.float32)
        mgroups = [ms_v[pl.ds(b * HISTP + s, LANES)] for s in group_starts]
        for l in range(HIST):
            g, lane = (divmod(l, LANES) if l < 48 else (3, l - 40))
            w = jnp.full((LANES,), mgroups[g][lane], dtype=jnp.float32)
            cnt = cnt + w
            for j in range(D // LANES):
                acc[j] = acc[j] + rbuf[2 * l + j // 2,
                                       (j % 2) * LANES:(j % 2 + 1) * LANES] * w
        inv = 1.0 / jnp.maximum(cnt, 1.0)
        for j in range(D // LANES):
            out_v[pl.ds(b * D + j * LANES, LANES)] = acc[j] * inv

    def body(g, carry):
        for ph in range(NBUF):
            b = g * NBUF + ph
            pltpu.make_async_copy(
                table_hbm.at[xp_v.at[pl.ds(b * HISTQ, 2 * HIST)]],
                rows[ph], sems[ph]).wait()
            compute_row(b, rows[ph])
            nxt = b + NBUF

            @pl.when(nxt < bpw)
            def _():
                issue(nxt, ph)
        return carry

    lax.fori_loop(0, bpw // NBUF, body, 0)
    pltpu.sync_copy(out_v, out_hbm.at[pl.ds(out_base, bpw * D)])


def kernel(x, mask, table):
    batch = x.shape[0]
    vocab = table.shape[0]
    bpw = batch // NW
    t2 = jnp.pad(table, ((0, 0), (0, DP - D))).reshape(3 * vocab, GW)
    x3 = x * 3
    xq = jnp.stack([x3, x3 + 1], axis=-1).reshape(batch, 2 * HIST)
    xp = jnp.pad(xq, ((0, 0), (0, HISTQ - 2 * HIST))).reshape(-1)
    ms = jnp.pad(mask, ((0, 0), (0, HISTP - HIST))).reshape(-1)

    mesh = plsc.VectorSubcoreMesh(core_axis_name="c", subcore_axis_name="s")
    tower = functools.partial(
        pl.kernel,
        out_type=jax.ShapeDtypeStruct((batch * D,), jnp.float32),
        mesh=mesh,
        scratch_types=[
            pltpu.VMEM((bpw * HISTQ,), jnp.int32),
            pltpu.VMEM((bpw * HISTP,), jnp.float32),
            pltpu.VMEM((2 * HIST, GW), jnp.float32),
            pltpu.VMEM((2 * HIST, GW), jnp.float32),
            pltpu.VMEM((2 * HIST, GW), jnp.float32),
            pltpu.VMEM((2 * HIST, GW), jnp.float32),
            pltpu.VMEM((bpw * D,), jnp.float32),
            pltpu.SemaphoreType.DMA,
            pltpu.SemaphoreType.DMA,
            pltpu.SemaphoreType.DMA,
            pltpu.SemaphoreType.DMA,
        ],
        compiler_params=pltpu.CompilerParams(use_tc_tiling_on_sc=False),
    )(functools.partial(_tower_body, bpw=bpw))

    return tower(xp, ms, t2).reshape(batch, D)


# speedup vs baseline: 1.8059x; 1.8059x over previous
import functools

import jax
import jax.numpy as jnp
from jax import lax
from jax.experimental import pallas as pl
from jax.experimental.pallas import tpu as pltpu
from jax.experimental.pallas import tpu_sc as plsc

D = 64
DP = 128
HIST = 50
HISTP = 56
LANES = 16
NC = 2
NS = 16
NW = NC * NS
NBUF = 4


def _tower_body(xp_hbm, ms_hbm, table_hbm, out_hbm,
                xp_v, ms_v, rows0, rows1, rows2, rows3, out_v,
                sem0, sem1, sem2, sem3, bpw):
    wid = lax.axis_index("s") * NC + lax.axis_index("c")
    rows = (rows0, rows1, rows2, rows3)
    sems = (sem0, sem1, sem2, sem3)
    in_base = wid * bpw * HISTP
    out_base = wid * bpw * D

    pltpu.sync_copy(xp_hbm.at[pl.ds(in_base, bpw * HISTP)], xp_v)
    pltpu.sync_copy(ms_hbm.at[pl.ds(in_base, bpw * HISTP)], ms_v)

    group_starts = (0, 16, 32, 40)

    def issue(b, ph):
        pltpu.async_copy(table_hbm.at[xp_v.at[pl.ds(b * HISTP, HIST)]],
                         rows[ph], sems[ph])

    for ph in range(NBUF):
        issue(ph, ph)

    def compute_row(b, rbuf):
        acc = [jnp.zeros((LANES,), jnp.float32) for _ in range(D // LANES)]
        cnt = jnp.zeros((LANES,), jnp.float32)
        mgroups = [ms_v[pl.ds(b * HISTP + s, LANES)] for s in group_starts]
        for l in range(HIST):
            g, lane = (divmod(l, LANES) if l < 48 else (3, l - 40))
            w = jnp.full((LANES,), mgroups[g][lane], dtype=jnp.float32)
            cnt = cnt + w
            for j in range(D // LANES):
                acc[j] = acc[j] + rbuf[l, j * LANES:(j + 1) * LANES] * w
        inv = 1.0 / jnp.maximum(cnt, 1.0)
        for j in range(D // LANES):
            out_v[pl.ds(b * D + j * LANES, LANES)] = acc[j] * inv

    def body(g, carry):
        for ph in range(NBUF):
            b = g * NBUF + ph
            pltpu.make_async_copy(
                table_hbm.at[xp_v.at[pl.ds(b * HISTP, HIST)]],
                rows[ph], sems[ph]).wait()
            compute_row(b, rows[ph])
            nxt = b + NBUF

            @pl.when(nxt < bpw)
            def _():
                issue(nxt, ph)
        return carry

    lax.fori_loop(0, bpw // NBUF, body, 0)
    pltpu.sync_copy(out_v, out_hbm.at[pl.ds(out_base, bpw * D)])


def kernel(x, mask, table):
    batch = x.shape[0]
    vocab = table.shape[0]
    bpw = batch // NW
    t2 = jnp.pad(table, ((0, 0), (0, DP - D))).reshape(2 * vocab, D)
    xp = jnp.pad(x << 1, ((0, 0), (0, HISTP - HIST))).reshape(-1)
    ms = jnp.pad(mask, ((0, 0), (0, HISTP - HIST))).reshape(-1)

    mesh = plsc.VectorSubcoreMesh(core_axis_name="c", subcore_axis_name="s")
    tower = functools.partial(
        pl.kernel,
        out_type=jax.ShapeDtypeStruct((batch * D,), jnp.float32),
        mesh=mesh,
        scratch_types=[
            pltpu.VMEM((bpw * HISTP,), jnp.int32),
            pltpu.VMEM((bpw * HISTP,), jnp.float32),
            pltpu.VMEM((HIST, D), jnp.float32),
            pltpu.VMEM((HIST, D), jnp.float32),
            pltpu.VMEM((HIST, D), jnp.float32),
            pltpu.VMEM((HIST, D), jnp.float32),
            pltpu.VMEM((bpw * D,), jnp.float32),
            pltpu.SemaphoreType.DMA,
            pltpu.SemaphoreType.DMA,
            pltpu.SemaphoreType.DMA,
            pltpu.SemaphoreType.DMA,
        ],
        compiler_params=pltpu.CompilerParams(use_tc_tiling_on_sc=False),
    )(functools.partial(_tower_body, bpw=bpw))

    return tower(xp, ms, t2).reshape(batch, D)
